# baseline (speedup 1.0000x reference)
import jax
import jax.numpy as jnp
from jax import lax
from jax.experimental import pallas as pl
from jax.experimental.pallas import tpu as pltpu
from jax.experimental.pallas import tpu_sc as plsc

_BLOCK_R = 1024


def _sc_gather_body(mid_hbm, tab_hbm, row_hbm, idx_v, row_v, sem):
    c = lax.axis_index("c")
    s = lax.axis_index("s")

    @pl.when(jnp.logical_and(c == 0, s == 0))
    def _():
        pltpu.sync_copy(mid_hbm, idx_v)
        pltpu.async_copy(tab_hbm.at[idx_v], row_v, sem).wait()
        pltpu.sync_copy(row_v, row_hbm)


def _sc_gather(mid, mod_emb_table):
    D = mod_emb_table.shape[1]
    mesh = plsc.VectorSubcoreMesh(core_axis_name="c", subcore_axis_name="s")
    return pl.kernel(
        _sc_gather_body,
        mesh=mesh,
        out_type=jax.ShapeDtypeStruct((1, D), mod_emb_table.dtype),
        scratch_types=[
            pltpu.VMEM((1,), jnp.int32),
            pltpu.VMEM((1, D), mod_emb_table.dtype),
            pltpu.SemaphoreType.DMA,
        ],
    )(mid, mod_emb_table)


def _tc_add_body(x_ref, row_ref, o_ref):
    o_ref[...] = x_ref[...] + row_ref[...]


def kernel(x, mod_emb_table, modality_id):
    B, S, D = x.shape
    R = B * S
    xf = x.reshape(R, D)
    mid = jnp.asarray(modality_id, jnp.int32).reshape(1)
    row = _sc_gather(mid, mod_emb_table)
    out = pl.pallas_call(
        _tc_add_body,
        grid=(R // _BLOCK_R,),
        in_specs=[
            pl.BlockSpec((_BLOCK_R, D), lambda i: (i, 0)),
            pl.BlockSpec((1, D), lambda i: (0, 0)),
        ],
        out_specs=pl.BlockSpec((_BLOCK_R, D), lambda i: (i, 0)),
        out_shape=jax.ShapeDtypeStruct((R, D), x.dtype),
    )(xf, row)
    return out.reshape(B, S, D)
